# baseline (speedup 1.0000x reference)
import functools

import jax
import jax.numpy as jnp
from jax import lax
from jax.experimental import pallas as pl
from jax.experimental.pallas import tpu as pltpu
from jax.experimental.pallas import tpu_sc as plsc

_K = 50
_BINS = 5
_PADK = 64
_CAP = 2048
_G = 128
_BIG = 1.5e38
_NW = 32


def _d2_body(r_blk, n, n_pad, pos_blk_ref, posT_ref, sqr_ref, sqc_ref,
             d2_ref, t_ref):
    i = pl.program_id(0)
    pos_r = pos_blk_ref[...]
    posT = posT_ref[...]
    sq_r = sqr_ref[...]
    sq_c = sqc_ref[...]
    dot = jax.lax.dot_general(pos_r, posT, (((1,), (0,)), ((), ())),
                              preferred_element_type=jnp.float32)
    d2 = jnp.maximum(sq_r + sq_c - 2.0 * dot, 0.0)
    col = jax.lax.broadcasted_iota(jnp.int32, (r_blk, n), 1)
    row_g = i * r_blk + jax.lax.broadcasted_iota(jnp.int32, (r_blk, n), 0)
    d2 = jnp.where(col == row_g, 1e10, d2)
    d2_ref[...] = d2

    pad = jnp.full((r_blk, n_pad - n), _BIG, jnp.float32)
    gm = jnp.min(jnp.concatenate([d2, pad], axis=1)
                 .reshape(r_blk, n_pad // _G, _G), axis=2)

    def tb(_, carry):
        g, _m = carry
        m = jnp.min(g, axis=1, keepdims=True)
        g = jnp.where(g == m, _BIG, g)
        return g, m

    _, t = jax.lax.fori_loop(0, _K, tb, (gm, jnp.zeros((r_blk, 1),
                                                       jnp.float32)))
    t_ref[...] = t


def _make_sc_select(n, rows_hi):
    mesh = plsc.VectorSubcoreMesh(core_axis_name="c", subcore_axis_name="s")

    @functools.partial(
        pl.kernel, mesh=mesh,
        compiler_params=pltpu.CompilerParams(needs_layout_passes=False),
        out_type=[jax.ShapeDtypeStruct((n * _PADK,), jnp.int32),
                  jax.ShapeDtypeStruct((n * _PADK,), jnp.float32)],
        scratch_types=[
            pltpu.VMEM((n,), jnp.float32),
            pltpu.VMEM((n,), jnp.float32),
            pltpu.VMEM((n,), jnp.float32),
            pltpu.VMEM((n,), jnp.float32),
            pltpu.VMEM((336,), jnp.float32),
            pltpu.VMEM((_CAP + 16,), jnp.float32),
            pltpu.VMEM((_CAP + 16,), jnp.int32),
            pltpu.VMEM((_CAP + 16,), jnp.float32),
            pltpu.VMEM((_CAP + 16,), jnp.int32),
        ],
    )
    def sc(d2_hbm, t_hbm, xs_hbm, ys_hbm, zs_hbm, nbr_hbm, val_hbm,
           row_v, xv, yv, zv, tv, cv, ci, ov, oi):
        w = lax.axis_index("s") * 2 + lax.axis_index("c")
        base = rows_hi * w - jnp.maximum(w - _NW // 2, 0)
        nrows = jnp.where(w < _NW // 2, rows_hi, rows_hi - 1)
        tbase = (base // 8) * 8
        toff = base - tbase
        pltpu.sync_copy(xs_hbm, xv)
        pltpu.sync_copy(ys_hbm, yv)
        pltpu.sync_copy(zs_hbm, zv)
        pltpu.sync_copy(t_hbm.at[pl.ds(tbase, 336)], tv)
        lanes = lax.broadcasted_iota(jnp.int32, (16,), 0)
        n_chunks = n // 16

        def row_body(rl, _):
            @pl.when(rl < nrows)
            def _():
                r = base + rl
                pltpu.sync_copy(d2_hbm.at[pl.ds(r * n, n)], row_v)
                tsplat = plsc.load_gather(
                    tv, [jnp.full((16,), rl + toff, jnp.int32)])

                def chunk(c, cnt):
                    v = row_v[pl.ds(c * 16, 16)]
                    m = v <= tsplat
                    off = jnp.minimum(cnt, _CAP)
                    plsc.store_compressed(cv.at[pl.ds(off, 16)], v, mask=m)
                    plsc.store_compressed(ci.at[pl.ds(off, 16)],
                                          lanes + c * 16, mask=m)
                    return jnp.minimum(cnt + jnp.sum(m.astype(jnp.int32)),
                                       _CAP)

                cnt = lax.fori_loop(0, n_chunks, chunk, 0)
                cv[pl.ds(cnt, 16)] = jnp.full((16,), _BIG, jnp.float32)
                nv = cnt // 16 + 1

                def wbody(oc):
                    def mn(vi, acc):
                        return jnp.minimum(acc, cv[pl.ds(vi * 16, 16)])
                    acc = lax.fori_loop(0, nv, mn,
                                        jnp.full((16,), _BIG, jnp.float32))
                    ms = jnp.full((16,), jnp.min(acc))

                    def ex(vi, oc):
                        v = cv[pl.ds(vi * 16, 16)]
                        m = v == ms
                        plsc.store_compressed(ov.at[pl.ds(oc, 16)], v, mask=m)
                        plsc.store_compressed(oi.at[pl.ds(oc, 16)],
                                              ci[pl.ds(vi * 16, 16)], mask=m)
                        cv[pl.ds(vi * 16, 16)] = jnp.where(
                            m, jnp.full((16,), _BIG, jnp.float32), v)
                        return oc + jnp.sum(m.astype(jnp.int32))

                    return lax.fori_loop(0, nv, ex, oc)

                lax.while_loop(lambda oc: oc < _K, wbody, 0)

                rsplat = jnp.full((16,), r, jnp.int32)
                xr = plsc.load_gather(xv, [rsplat])
                yr = plsc.load_gather(yv, [rsplat])
                zr = plsc.load_gather(zv, [rsplat])

                def gd(vi, _):
                    idx = jnp.clip(oi[pl.ds(vi * 16, 16)], 0, n - 1)
                    dx = plsc.load_gather(xv, [idx]) - xr
                    dy = plsc.load_gather(yv, [idx]) - yr
                    dz = plsc.load_gather(zv, [idx]) - zr
                    ov[pl.ds(vi * 16, 16)] = dx * dx + dy * dy + dz * dz
                    return 0

                lax.fori_loop(0, _PADK // 16, gd, 0)
                pltpu.sync_copy(oi.at[pl.ds(0, _PADK)],
                                nbr_hbm.at[pl.ds(r * _PADK, _PADK)])
                pltpu.sync_copy(ov.at[pl.ds(0, _PADK)],
                                val_hbm.at[pl.ds(r * _PADK, _PADK)])
            return 0

        lax.fori_loop(0, rows_hi, row_body, 0)

    return sc


def _dist_body(vals_ref, dist_ref, rmax_ref):
    vals = vals_ref[...]
    dist = jnp.sqrt(vals[:, :_K] + 1e-12)
    dist_ref[...] = dist
    rmax_ref[...] = jnp.max(dist, axis=1, keepdims=True)


def _rbf_body(cut_ref, dist_ref, out_ref):
    c = cut_ref[0]
    delta = c * 0.25
    sigma = delta + 1e-9
    inv = 1.0 / (2.0 * sigma * sigma)
    d = dist_ref[...]
    centers = jax.lax.broadcasted_iota(
        jnp.int32, (1, _BINS), 1).astype(jnp.float32) * delta
    out_ref[...] = jnp.exp(-((d - centers) ** 2) * inv)


def kernel(pos):
    n = pos.shape[0]
    r_blk = next(r for r in (200, 100, 40, 8, 1) if n % r == 0)
    n_pad = ((n + _G - 1) // _G) * _G
    posT = pos.T
    sq = jnp.sum(pos * pos, axis=1)

    d2, t = pl.pallas_call(
        lambda *refs: _d2_body(r_blk, n, n_pad, *refs),
        grid=(n // r_blk,),
        in_specs=[
            pl.BlockSpec((r_blk, 3), lambda i: (i, 0)),
            pl.BlockSpec((3, n), lambda i: (0, 0)),
            pl.BlockSpec((r_blk, 1), lambda i: (i, 0)),
            pl.BlockSpec((1, n), lambda i: (0, 0)),
        ],
        out_specs=[
            pl.BlockSpec((r_blk, n), lambda i: (i, 0)),
            pl.BlockSpec((r_blk, 1), lambda i: (i, 0)),
        ],
        out_shape=[
            jax.ShapeDtypeStruct((n, n), jnp.float32),
            jax.ShapeDtypeStruct((n, 1), jnp.float32),
        ],
    )(pos, posT, sq.reshape(n, 1), sq.reshape(1, n))

    t_padded = jnp.pad(t.reshape(-1), (0, 336))
    rows_hi = -(-n // _NW)
    nbr, vals = _make_sc_select(n, rows_hi)(
        d2.reshape(-1), t_padded, pos[:, 0], pos[:, 1], pos[:, 2])
    nbr = nbr.reshape(n, _PADK)
    vals = vals.reshape(n, _PADK)

    dist, rmax = pl.pallas_call(
        _dist_body,
        grid=(n // r_blk,),
        in_specs=[pl.BlockSpec((r_blk, _PADK), lambda i: (i, 0))],
        out_specs=[
            pl.BlockSpec((r_blk, _K), lambda i: (i, 0)),
            pl.BlockSpec((r_blk, 1), lambda i: (i, 0)),
        ],
        out_shape=[
            jax.ShapeDtypeStruct((n, _K), jnp.float32),
            jax.ShapeDtypeStruct((n, 1), jnp.float32),
        ],
    )(vals)

    cutoff = jnp.max(rmax).reshape(1)

    e = n * _K
    b_blk = next(b for b in (20000, 5000, 1000, 200, 50) if e % b == 0)
    ea_half = pl.pallas_call(
        _rbf_body,
        grid=(e // b_blk,),
        in_specs=[
            pl.BlockSpec(memory_space=pltpu.SMEM),
            pl.BlockSpec((b_blk, 1), lambda j: (j, 0)),
        ],
        out_specs=pl.BlockSpec((b_blk, _BINS), lambda j: (j, 0)),
        out_shape=jax.ShapeDtypeStruct((e, _BINS), jnp.float32),
    )(cutoff, dist.reshape(e, 1))

    src = nbr[:, :_K].reshape(-1)
    dst = jnp.broadcast_to(
        jnp.arange(n, dtype=jnp.int32)[:, None], (n, _K)).reshape(-1)
    edge_index = jnp.stack([
        jnp.concatenate([src, dst]),
        jnp.concatenate([dst, src]),
    ])
    edge_attr = jnp.concatenate([ea_half, ea_half], axis=0)
    return edge_index, edge_attr


# speedup vs baseline: 2.8822x; 2.8822x over previous
import functools

import jax
import jax.numpy as jnp
from jax import lax
from jax.experimental import pallas as pl
from jax.experimental.pallas import tpu as pltpu
from jax.experimental.pallas import tpu_sc as plsc

_K = 50
_BINS = 5
_PADK = 64
_CAP = 2048
_G = 128
_BIG = 1.5e38
_NW = 32


def _d2_body(r_blk, n, n_pad, pos_blk_ref, posT_ref, sqr_ref, sqc_ref,
             d2_ref, t_ref):
    i = pl.program_id(0)
    pos_r = pos_blk_ref[...]
    posT = posT_ref[...]
    sq_r = sqr_ref[...]
    sq_c = sqc_ref[...]
    dot = jax.lax.dot_general(pos_r, posT, (((1,), (0,)), ((), ())),
                              preferred_element_type=jnp.float32)
    d2 = jnp.maximum(sq_r + sq_c - 2.0 * dot, 0.0)
    col = jax.lax.broadcasted_iota(jnp.int32, (r_blk, n), 1)
    row_g = i * r_blk + jax.lax.broadcasted_iota(jnp.int32, (r_blk, n), 0)
    d2 = jnp.where(col == row_g, 1e10, d2)
    d2_ref[...] = d2

    pad = jnp.full((r_blk, n_pad - n), _BIG, jnp.float32)
    gm = jnp.min(jnp.concatenate([d2, pad], axis=1)
                 .reshape(r_blk, n_pad // _G, _G), axis=2)
    gmT = gm.T

    def tb(_, carry):
        g, _m = carry
        m = jnp.min(g, axis=0, keepdims=True)
        g = jnp.where(g == m, _BIG, g)
        return g, m

    _, t = jax.lax.fori_loop(0, _K, tb, (gmT, jnp.zeros((1, r_blk),
                                                        jnp.float32)))
    t_ref[...] = t.T


def _make_sc_select(n, rows_hi):
    mesh = plsc.VectorSubcoreMesh(core_axis_name="c", subcore_axis_name="s")

    @functools.partial(
        pl.kernel, mesh=mesh,
        compiler_params=pltpu.CompilerParams(needs_layout_passes=False),
        out_type=[jax.ShapeDtypeStruct((n * _PADK,), jnp.int32),
                  jax.ShapeDtypeStruct((n * _PADK,), jnp.float32)],
        scratch_types=[
            pltpu.VMEM((n,), jnp.float32),
            pltpu.VMEM((n,), jnp.float32),
            pltpu.VMEM((n,), jnp.float32),
            pltpu.VMEM((n,), jnp.float32),
            pltpu.VMEM((336,), jnp.float32),
            pltpu.VMEM((_CAP + 16,), jnp.float32),
            pltpu.VMEM((_CAP + 16,), jnp.int32),
            pltpu.VMEM((_CAP + 16,), jnp.float32),
            pltpu.VMEM((_CAP + 16,), jnp.int32),
        ],
    )
    def sc(d2_hbm, t_hbm, xs_hbm, ys_hbm, zs_hbm, nbr_hbm, val_hbm,
           row_v, xv, yv, zv, tv, cv, ci, ov, oi):
        w = lax.axis_index("s") * 2 + lax.axis_index("c")
        base = rows_hi * w - jnp.maximum(w - _NW // 2, 0)
        nrows = jnp.where(w < _NW // 2, rows_hi, rows_hi - 1)
        tbase = (base // 8) * 8
        toff = base - tbase
        pltpu.sync_copy(xs_hbm, xv)
        pltpu.sync_copy(ys_hbm, yv)
        pltpu.sync_copy(zs_hbm, zv)
        pltpu.sync_copy(t_hbm.at[pl.ds(tbase, 336)], tv)
        lanes = lax.broadcasted_iota(jnp.int32, (16,), 0)
        n_chunks = n // 16

        def row_body(rl, _):
            @pl.when(rl < nrows)
            def _():
                r = base + rl
                pltpu.sync_copy(d2_hbm.at[pl.ds(r * n, n)], row_v)
                tsplat = plsc.load_gather(
                    tv, [jnp.full((16,), rl + toff, jnp.int32)])

                def chunk(c, cnt):
                    v = row_v[pl.ds(c * 16, 16)]
                    m = v <= tsplat
                    off = jnp.minimum(cnt, _CAP)
                    plsc.store_compressed(cv.at[pl.ds(off, 16)], v, mask=m)
                    plsc.store_compressed(ci.at[pl.ds(off, 16)],
                                          lanes + c * 16, mask=m)
                    return jnp.minimum(cnt + jnp.sum(m.astype(jnp.int32)),
                                       _CAP)

                cnt = lax.fori_loop(0, n_chunks, chunk, 0)
                cv[pl.ds(cnt, 16)] = jnp.full((16,), _BIG, jnp.float32)
                nv = cnt // 16 + 1

                def wbody(oc):
                    def mn(vi, acc):
                        return jnp.minimum(acc, cv[pl.ds(vi * 16, 16)])
                    acc = lax.fori_loop(0, nv, mn,
                                        jnp.full((16,), _BIG, jnp.float32))
                    ms = jnp.full((16,), jnp.min(acc))

                    def ex(vi, oc):
                        v = cv[pl.ds(vi * 16, 16)]
                        m = v == ms
                        plsc.store_compressed(ov.at[pl.ds(oc, 16)], v, mask=m)
                        plsc.store_compressed(oi.at[pl.ds(oc, 16)],
                                              ci[pl.ds(vi * 16, 16)], mask=m)
                        cv[pl.ds(vi * 16, 16)] = jnp.where(
                            m, jnp.full((16,), _BIG, jnp.float32), v)
                        return oc + jnp.sum(m.astype(jnp.int32))

                    return lax.fori_loop(0, nv, ex, oc)

                lax.while_loop(lambda oc: oc < _K, wbody, 0)

                rsplat = jnp.full((16,), r, jnp.int32)
                xr = plsc.load_gather(xv, [rsplat])
                yr = plsc.load_gather(yv, [rsplat])
                zr = plsc.load_gather(zv, [rsplat])

                def gd(vi, _):
                    idx = jnp.clip(oi[pl.ds(vi * 16, 16)], 0, n - 1)
                    dx = plsc.load_gather(xv, [idx]) - xr
                    dy = plsc.load_gather(yv, [idx]) - yr
                    dz = plsc.load_gather(zv, [idx]) - zr
                    ov[pl.ds(vi * 16, 16)] = dx * dx + dy * dy + dz * dz
                    return 0

                lax.fori_loop(0, _PADK // 16, gd, 0)
                pltpu.sync_copy(oi.at[pl.ds(0, _PADK)],
                                nbr_hbm.at[pl.ds(r * _PADK, _PADK)])
                pltpu.sync_copy(ov.at[pl.ds(0, _PADK)],
                                val_hbm.at[pl.ds(r * _PADK, _PADK)])
            return 0

        lax.fori_loop(0, rows_hi, row_body, 0)

    return sc


def _dist_body(vals_ref, dist_ref, rmax_ref):
    vals = vals_ref[...]
    dist = jnp.sqrt(vals[:, :_K] + 1e-12)
    dist_ref[...] = dist
    rmax_ref[...] = jnp.max(dist, axis=1, keepdims=True)


def _rbf_body(cut_ref, dist_ref, out_ref):
    c = cut_ref[0]
    delta = c * 0.25
    sigma = delta + 1e-9
    inv = 1.0 / (2.0 * sigma * sigma)
    d = dist_ref[...]
    centers = jax.lax.broadcasted_iota(
        jnp.int32, (1, _BINS), 1).astype(jnp.float32) * delta
    out_ref[...] = jnp.exp(-((d - centers) ** 2) * inv)


def kernel(pos):
    n = pos.shape[0]
    r_blk = next(r for r in (200, 100, 40, 8, 1) if n % r == 0)
    n_pad = ((n + _G - 1) // _G) * _G
    posT = pos.T
    sq = jnp.sum(pos * pos, axis=1)

    d2, t = pl.pallas_call(
        lambda *refs: _d2_body(r_blk, n, n_pad, *refs),
        grid=(n // r_blk,),
        in_specs=[
            pl.BlockSpec((r_blk, 3), lambda i: (i, 0)),
            pl.BlockSpec((3, n), lambda i: (0, 0)),
            pl.BlockSpec((r_blk, 1), lambda i: (i, 0)),
            pl.BlockSpec((1, n), lambda i: (0, 0)),
        ],
        out_specs=[
            pl.BlockSpec((r_blk, n), lambda i: (i, 0)),
            pl.BlockSpec((r_blk, 1), lambda i: (i, 0)),
        ],
        out_shape=[
            jax.ShapeDtypeStruct((n, n), jnp.float32),
            jax.ShapeDtypeStruct((n, 1), jnp.float32),
        ],
    )(pos, posT, sq.reshape(n, 1), sq.reshape(1, n))

    t_padded = jnp.pad(t.reshape(-1), (0, 336))
    rows_hi = -(-n // _NW)
    nbr, vals = _make_sc_select(n, rows_hi)(
        d2.reshape(-1), t_padded, pos[:, 0], pos[:, 1], pos[:, 2])
    nbr = nbr.reshape(n, _PADK)
    vals = vals.reshape(n, _PADK)

    dist, rmax = pl.pallas_call(
        _dist_body,
        grid=(n // r_blk,),
        in_specs=[pl.BlockSpec((r_blk, _PADK), lambda i: (i, 0))],
        out_specs=[
            pl.BlockSpec((r_blk, _K), lambda i: (i, 0)),
            pl.BlockSpec((r_blk, 1), lambda i: (i, 0)),
        ],
        out_shape=[
            jax.ShapeDtypeStruct((n, _K), jnp.float32),
            jax.ShapeDtypeStruct((n, 1), jnp.float32),
        ],
    )(vals)

    cutoff = jnp.max(rmax).reshape(1)

    e = n * _K
    b_blk = next(b for b in (20000, 5000, 1000, 200, 50) if e % b == 0)
    ea_half = pl.pallas_call(
        _rbf_body,
        grid=(e // b_blk,),
        in_specs=[
            pl.BlockSpec(memory_space=pltpu.SMEM),
            pl.BlockSpec((b_blk, 1), lambda j: (j, 0)),
        ],
        out_specs=pl.BlockSpec((b_blk, _BINS), lambda j: (j, 0)),
        out_shape=jax.ShapeDtypeStruct((e, _BINS), jnp.float32),
    )(cutoff, dist.reshape(e, 1))

    src = nbr[:, :_K].reshape(-1)
    dst = jnp.broadcast_to(
        jnp.arange(n, dtype=jnp.int32)[:, None], (n, _K)).reshape(-1)
    edge_index = jnp.stack([
        jnp.concatenate([src, dst]),
        jnp.concatenate([dst, src]),
    ])
    edge_attr = jnp.concatenate([ea_half, ea_half], axis=0)
    return edge_index, edge_attr


# speedup vs baseline: 3.2110x; 1.1141x over previous
import functools

import jax
import jax.numpy as jnp
from jax import lax
from jax.experimental import pallas as pl
from jax.experimental.pallas import tpu as pltpu
from jax.experimental.pallas import tpu_sc as plsc

_K = 50
_BINS = 5
_PADK = 64
_CAP = 2048
_G = 128
_BIG = 1.5e38
_NW = 32


def _d2_body(r_blk, n, n_pad, pos_blk_ref, posT_ref, sqr_ref, sqc_ref,
             d2_ref, t_ref):
    i = pl.program_id(0)
    pos_r = pos_blk_ref[...]
    posT = posT_ref[...]
    sq_r = sqr_ref[...]
    sq_c = sqc_ref[...]
    dot = jax.lax.dot_general(pos_r, posT, (((1,), (0,)), ((), ())),
                              preferred_element_type=jnp.float32)
    d2 = jnp.maximum(sq_r + sq_c - 2.0 * dot, 0.0)
    col = jax.lax.broadcasted_iota(jnp.int32, (r_blk, n), 1)
    row_g = i * r_blk + jax.lax.broadcasted_iota(jnp.int32, (r_blk, n), 0)
    d2 = jnp.where(col == row_g, 1e10, d2)
    d2_ref[...] = d2

    pad = jnp.full((r_blk, n_pad - n), _BIG, jnp.float32)
    gm = jnp.min(jnp.concatenate([d2, pad], axis=1)
                 .reshape(r_blk, n_pad // _G, _G), axis=2)
    gmT = gm.T

    def tb(_, carry):
        g, _m = carry
        m = jnp.min(g, axis=0, keepdims=True)
        g = jnp.where(g == m, _BIG, g)
        return g, m

    _, t = jax.lax.fori_loop(0, _K, tb, (gmT, jnp.zeros((1, r_blk),
                                                        jnp.float32)))
    t_ref[...] = t.T


def _make_sc_select(n, rows_hi):
    mesh = plsc.VectorSubcoreMesh(core_axis_name="c", subcore_axis_name="s")

    @functools.partial(
        pl.kernel, mesh=mesh,
        compiler_params=pltpu.CompilerParams(needs_layout_passes=False),
        out_type=[jax.ShapeDtypeStruct((n * _PADK,), jnp.int32),
                  jax.ShapeDtypeStruct((n * _PADK,), jnp.float32)],
        scratch_types=[
            pltpu.VMEM((n,), jnp.float32),
            pltpu.VMEM((n,), jnp.float32),
            pltpu.VMEM((n,), jnp.float32),
            pltpu.VMEM((n,), jnp.float32),
            pltpu.VMEM((336,), jnp.float32),
            pltpu.VMEM((_CAP + 16,), jnp.float32),
            pltpu.VMEM((_CAP + 16,), jnp.int32),
            pltpu.VMEM((_CAP + 16,), jnp.float32),
            pltpu.VMEM((_CAP + 16,), jnp.int32),
        ],
    )
    def sc(d2_hbm, t_hbm, xs_hbm, ys_hbm, zs_hbm, nbr_hbm, val_hbm,
           row_v, xv, yv, zv, tv, cv, ci, ov, oi):
        w = lax.axis_index("s") * 2 + lax.axis_index("c")
        base = rows_hi * w - jnp.maximum(w - _NW // 2, 0)
        nrows = jnp.where(w < _NW // 2, rows_hi, rows_hi - 1)
        tbase = (base // 8) * 8
        toff = base - tbase
        pltpu.sync_copy(xs_hbm, xv)
        pltpu.sync_copy(ys_hbm, yv)
        pltpu.sync_copy(zs_hbm, zv)
        pltpu.sync_copy(t_hbm.at[pl.ds(tbase, 336)], tv)
        lanes = lax.broadcasted_iota(jnp.int32, (16,), 0)
        n_chunks = n // 16

        def row_body(rl, _):
            @pl.when(rl < nrows)
            def _():
                r = base + rl
                pltpu.sync_copy(d2_hbm.at[pl.ds(r * n, n)], row_v)
                tsplat = plsc.load_gather(
                    tv, [jnp.full((16,), rl + toff, jnp.int32)])

                def chunk(c, cnt):
                    b = c * 64
                    vs = [row_v[pl.ds(b + 16 * j, 16)] for j in range(4)]
                    ms = [v <= tsplat for v in vs]
                    ps = [jnp.sum(m.astype(jnp.int32)) for m in ms]
                    tot = ps[0] + ps[1] + ps[2] + ps[3]

                    @pl.when(tot > 0)
                    def _():
                        off = cnt
                        for j in range(4):
                            o = jnp.minimum(off, _CAP)
                            plsc.store_compressed(
                                cv.at[pl.ds(o, 16)], vs[j], mask=ms[j])
                            plsc.store_compressed(
                                ci.at[pl.ds(o, 16)], lanes + (b + 16 * j),
                                mask=ms[j])
                            off = off + ps[j]

                    return jnp.minimum(cnt + tot, _CAP)

                cnt = lax.fori_loop(0, n_chunks // 4, chunk, 0)
                for j in range(n_chunks // 4 * 4, n_chunks):
                    vt = row_v[pl.ds(j * 16, 16)]
                    mt = vt <= tsplat
                    ot = jnp.minimum(cnt, _CAP)
                    plsc.store_compressed(cv.at[pl.ds(ot, 16)], vt, mask=mt)
                    plsc.store_compressed(ci.at[pl.ds(ot, 16)],
                                          lanes + j * 16, mask=mt)
                    cnt = jnp.minimum(cnt + jnp.sum(mt.astype(jnp.int32)),
                                      _CAP)
                cv[pl.ds(cnt, 16)] = jnp.full((16,), _BIG, jnp.float32)
                nv = cnt // 16 + 1

                def wbody(oc):
                    def mn(vi, acc):
                        return jnp.minimum(acc, cv[pl.ds(vi * 16, 16)])
                    acc = lax.fori_loop(0, nv, mn,
                                        jnp.full((16,), _BIG, jnp.float32))
                    ms = jnp.full((16,), jnp.min(acc))

                    def ex(vi, oc):
                        v = cv[pl.ds(vi * 16, 16)]
                        m = v == ms
                        plsc.store_compressed(ov.at[pl.ds(oc, 16)], v, mask=m)
                        plsc.store_compressed(oi.at[pl.ds(oc, 16)],
                                              ci[pl.ds(vi * 16, 16)], mask=m)
                        cv[pl.ds(vi * 16, 16)] = jnp.where(
                            m, jnp.full((16,), _BIG, jnp.float32), v)
                        return oc + jnp.sum(m.astype(jnp.int32))

                    return lax.fori_loop(0, nv, ex, oc)

                lax.while_loop(lambda oc: oc < _K, wbody, 0)

                rsplat = jnp.full((16,), r, jnp.int32)
                xr = plsc.load_gather(xv, [rsplat])
                yr = plsc.load_gather(yv, [rsplat])
                zr = plsc.load_gather(zv, [rsplat])

                def gd(vi, _):
                    idx = jnp.clip(oi[pl.ds(vi * 16, 16)], 0, n - 1)
                    dx = plsc.load_gather(xv, [idx]) - xr
                    dy = plsc.load_gather(yv, [idx]) - yr
                    dz = plsc.load_gather(zv, [idx]) - zr
                    ov[pl.ds(vi * 16, 16)] = dx * dx + dy * dy + dz * dz
                    return 0

                lax.fori_loop(0, _PADK // 16, gd, 0)
                pltpu.sync_copy(oi.at[pl.ds(0, _PADK)],
                                nbr_hbm.at[pl.ds(r * _PADK, _PADK)])
                pltpu.sync_copy(ov.at[pl.ds(0, _PADK)],
                                val_hbm.at[pl.ds(r * _PADK, _PADK)])
            return 0

        lax.fori_loop(0, rows_hi, row_body, 0)

    return sc


def _dist_body(vals_ref, dist_ref, rmax_ref):
    vals = vals_ref[...]
    dist = jnp.sqrt(vals[:, :_K] + 1e-12)
    dist_ref[...] = dist
    rmax_ref[...] = jnp.max(dist, axis=1, keepdims=True)


def _rbf_body(cut_ref, dist_ref, out_ref):
    c = cut_ref[0]
    delta = c * 0.25
    sigma = delta + 1e-9
    inv = 1.0 / (2.0 * sigma * sigma)
    d = dist_ref[...]
    centers = jax.lax.broadcasted_iota(
        jnp.int32, (1, _BINS), 1).astype(jnp.float32) * delta
    out_ref[...] = jnp.exp(-((d - centers) ** 2) * inv)


def kernel(pos):
    n = pos.shape[0]
    r_blk = next(r for r in (200, 100, 40, 8, 1) if n % r == 0)
    n_pad = ((n + _G - 1) // _G) * _G
    posT = pos.T
    sq = jnp.sum(pos * pos, axis=1)

    d2, t = pl.pallas_call(
        lambda *refs: _d2_body(r_blk, n, n_pad, *refs),
        grid=(n // r_blk,),
        in_specs=[
            pl.BlockSpec((r_blk, 3), lambda i: (i, 0)),
            pl.BlockSpec((3, n), lambda i: (0, 0)),
            pl.BlockSpec((r_blk, 1), lambda i: (i, 0)),
            pl.BlockSpec((1, n), lambda i: (0, 0)),
        ],
        out_specs=[
            pl.BlockSpec((r_blk, n), lambda i: (i, 0)),
            pl.BlockSpec((r_blk, 1), lambda i: (i, 0)),
        ],
        out_shape=[
            jax.ShapeDtypeStruct((n, n), jnp.float32),
            jax.ShapeDtypeStruct((n, 1), jnp.float32),
        ],
    )(pos, posT, sq.reshape(n, 1), sq.reshape(1, n))

    t_padded = jnp.pad(t.reshape(-1), (0, 336))
    rows_hi = -(-n // _NW)
    nbr, vals = _make_sc_select(n, rows_hi)(
        d2.reshape(-1), t_padded, pos[:, 0], pos[:, 1], pos[:, 2])
    nbr = nbr.reshape(n, _PADK)
    vals = vals.reshape(n, _PADK)

    dist, rmax = pl.pallas_call(
        _dist_body,
        grid=(n // r_blk,),
        in_specs=[pl.BlockSpec((r_blk, _PADK), lambda i: (i, 0))],
        out_specs=[
            pl.BlockSpec((r_blk, _K), lambda i: (i, 0)),
            pl.BlockSpec((r_blk, 1), lambda i: (i, 0)),
        ],
        out_shape=[
            jax.ShapeDtypeStruct((n, _K), jnp.float32),
            jax.ShapeDtypeStruct((n, 1), jnp.float32),
        ],
    )(vals)

    cutoff = jnp.max(rmax).reshape(1)

    e = n * _K
    b_blk = next(b for b in (20000, 5000, 1000, 200, 50) if e % b == 0)
    ea_half = pl.pallas_call(
        _rbf_body,
        grid=(e // b_blk,),
        in_specs=[
            pl.BlockSpec(memory_space=pltpu.SMEM),
            pl.BlockSpec((b_blk, 1), lambda j: (j, 0)),
        ],
        out_specs=pl.BlockSpec((b_blk, _BINS), lambda j: (j, 0)),
        out_shape=jax.ShapeDtypeStruct((e, _BINS), jnp.float32),
    )(cutoff, dist.reshape(e, 1))

    src = nbr[:, :_K].reshape(-1)
    dst = jnp.broadcast_to(
        jnp.arange(n, dtype=jnp.int32)[:, None], (n, _K)).reshape(-1)
    edge_index = jnp.stack([
        jnp.concatenate([src, dst]),
        jnp.concatenate([dst, src]),
    ])
    edge_attr = jnp.concatenate([ea_half, ea_half], axis=0)
    return edge_index, edge_attr
